# baseline (speedup 1.0000x reference)
import jax
import jax.numpy as jnp
from jax.experimental import pallas as pl

_K = 20
_BITS = 31


def _body(x_ref, o_ref):
    x = x_ref[...]
    rows = x.shape[0]
    xi = jax.lax.bitcast_convert_type(x, jnp.int32)


    hi = (xi >> 16).astype(jnp.int16)
    h_star = jnp.zeros((rows, 1), jnp.int32)
    for b in range(14, -1, -1):
        mid = h_star + (1 << b)
        n = jnp.sum(hi < mid.astype(jnp.int16), axis=1, keepdims=True,
                    dtype=jnp.int32)
        h_star = jnp.where(n <= _K, mid, h_star)
    h16 = h_star.astype(jnp.int16)
    n_base = jnp.sum(hi < h16, axis=1, keepdims=True, dtype=jnp.int32)

    lo_s = ((xi & 0xFFFF) - 32768).astype(jnp.int16)
    z = jnp.where(hi == h16, lo_s, jnp.int16(32767))
    lo_u = jnp.zeros((rows, 1), jnp.int32)
    for b in range(15, -1, -1):
        mid_u = lo_u + (1 << b)
        mid_s = (mid_u - 32768).astype(jnp.int16)
        n = n_base + jnp.sum(z < mid_s, axis=1, keepdims=True, dtype=jnp.int32)
        lo_u = jnp.where(n <= _K, mid_u, lo_u)

    t_bits = (h_star << 16) | lo_u
    t = jax.lax.bitcast_convert_type(t_bits, jnp.float32)

    less = x < t
    c_less = jnp.sum(less, axis=1, keepdims=True, dtype=jnp.int32)
    s_less = jnp.sum(jnp.where(less, x, 0.0), axis=1, keepdims=True)
    sum_k = s_less + (jnp.float32(_K) - c_less.astype(jnp.float32)) * t

    tk = t + jnp.float32(1e-10)
    inv = 1.0 / (jnp.float32(_K) * tk - sum_k)
    o_ref[...] = jnp.maximum((tk - x) * inv, 0.0)


def kernel(distances):
    n_rows, n_cols = distances.shape
    block_rows = 256 if n_rows % 256 == 0 else n_rows
    grid = (n_rows // block_rows,)
    return pl.pallas_call(
        _body,
        grid=grid,
        in_specs=[pl.BlockSpec((block_rows, n_cols), lambda i: (i, 0))],
        out_specs=pl.BlockSpec((block_rows, n_cols), lambda i: (i, 0)),
        out_shape=jax.ShapeDtypeStruct((n_rows, n_cols), jnp.float32),
    )(distances)


# speedup vs baseline: 3.6608x; 3.6608x over previous
import jax
import jax.numpy as jnp
from jax.experimental import pallas as pl
from jax.experimental.pallas import tpu as pltpu

_K = 20


def _packed_count(mask01, acc_base):
    pair = pltpu.bitcast(mask01, jnp.int32)
    return acc_base + jnp.sum(pair, axis=1, keepdims=True, dtype=jnp.int32)


def _body(x_ref, o_ref):
    x = x_ref[...]
    rows = x.shape[0]
    xi = jax.lax.bitcast_convert_type(x, jnp.int32)
    zero_pair = jnp.zeros((rows // 2, 1), jnp.int32)
    one = jnp.int16(1)
    nil = jnp.int16(0)

    hi = (xi >> 16).astype(jnp.int16)
    st1 = zero_pair
    for b in range(14, -1, -1):
        mid_pair = st1 + ((1 << b) | (1 << (b + 16)))
        mid16 = pltpu.bitcast(mid_pair, jnp.int16)
        n = _packed_count(jnp.where(hi < mid16, one, nil), zero_pair)
        go_a = ((n >> 16) <= _K).astype(jnp.int32) << (b + 16)
        go_b = ((n & 0xFFFF) <= _K).astype(jnp.int32) << b
        st1 = st1 + go_a + go_b
    h16 = pltpu.bitcast(st1, jnp.int16)

    n_base = _packed_count(jnp.where(hi < h16, one, nil), zero_pair)

    lo_s = ((xi & 0xFFFF) - 32768).astype(jnp.int16)
    z = jnp.where(hi == h16, lo_s, jnp.int16(32767))
    st2 = zero_pair
    for b in range(15, -1, -1):
        inc = (1 << b) | (1 << (b + 16))
        if inc >= 2**31:
            inc -= 2**32
        mid_pair = st2 + inc
        mid16 = pltpu.bitcast(mid_pair, jnp.int16) ^ jnp.int16(-32768)
        n = _packed_count(jnp.where(z < mid16, one, nil), n_base)
        go_a = (((n >> 16) & 0xFFFF) <= _K).astype(jnp.int32) << (b + 16)
        go_b = ((n & 0xFFFF) <= _K).astype(jnp.int32) << b
        st2 = st2 + go_a + go_b
    lo16 = pltpu.bitcast(st2, jnp.int16)

    t_bits = (h16.astype(jnp.int32) << 16) | (lo16.astype(jnp.int32) & 0xFFFF)
    t = jax.lax.bitcast_convert_type(t_bits, jnp.float32)

    less = x < t
    c_less = jnp.sum(less, axis=1, keepdims=True, dtype=jnp.int32)
    s_less = jnp.sum(jnp.where(less, x, 0.0), axis=1, keepdims=True)
    sum_k = s_less + (jnp.float32(_K) - c_less.astype(jnp.float32)) * t

    tk = t + jnp.float32(1e-10)
    inv = 1.0 / (jnp.float32(_K) * tk - sum_k)
    o_ref[...] = jnp.maximum((tk - x) * inv, 0.0)


def kernel(distances):
    n_rows, n_cols = distances.shape
    block_rows = 256 if n_rows % 256 == 0 else n_rows
    grid = (n_rows // block_rows,)
    return pl.pallas_call(
        _body,
        grid=grid,
        in_specs=[pl.BlockSpec((block_rows, n_cols), lambda i: (i, 0))],
        out_specs=pl.BlockSpec((block_rows, n_cols), lambda i: (i, 0)),
        out_shape=jax.ShapeDtypeStruct((n_rows, n_cols), jnp.float32),
    )(distances)
